# Initial kernel scaffold; baseline (speedup 1.0000x reference)
import functools

import jax
import jax.numpy as jnp
from jax import lax
from jax.experimental import pallas as pl
from jax.experimental.pallas import tpu as pltpu
from jax.experimental.pallas import tpu_sc as plsc

_N = 100000
_NU = 50000
_D = 32
_H = 16
_E = 1600000
_NSUB = 16
_CH = 4096
_K = _CH // 128
_ROUNDS = 25
_EPAD = _NSUB * _CH * _ROUNDS
_RPS = _N // _NSUB


def _splat(vec, lane):
    idx = jnp.full((16, 1), lane, dtype=jnp.int32)
    dn = lax.GatherDimensionNumbers(
        offset_dims=(), collapsed_slice_dims=(0,), start_index_map=(0,))
    return lax.gather(vec, idx, dn, (1,),
                      mode=lax.GatherScatterMode.PROMISE_IN_BOUNDS)


def _sc_layer(t_in, src2, dst2, w2):
    mesh = plsc.VectorSubcoreMesh(core_axis_name="c", subcore_axis_name="s")

    @functools.partial(
        pl.kernel,
        out_type=jax.ShapeDtypeStruct((2 * _N, _H), jnp.float32),
        mesh=mesh,
        scratch_types=[
            pltpu.VMEM((_K, 128), jnp.int32),
            pltpu.VMEM((_K, 128), jnp.int32),
            pltpu.VMEM((_CH // 16, 16), jnp.float32),
            pltpu.VMEM((_CH, _H), jnp.float32),
            pltpu.VMEM_SHARED((_N, _H), jnp.float32),
            pltpu.SemaphoreType.DMA,
            pltpu.SemaphoreType.DMA,
        ],
    )
    def layer(t_hbm, src_hbm, dst_hbm, w_hbm, out_hbm,
              gidx_v, didx_v, w_v, rows_v, acc, gsem, ssem):
        cc = lax.axis_index("c")
        ss = lax.axis_index("s")

        zero = jnp.zeros((16,), jnp.float32)

        @pl.loop(0, _CH // 16)
        def _(g):
            for m in range(16):
                rows_v[g * 16 + m] = zero

        pltpu.sync_copy(rows_v, acc.at[pl.ds(ss * _RPS, _CH)])
        pltpu.sync_copy(rows_v.at[pl.ds(0, _RPS - _CH)],
                        acc.at[pl.ds(ss * _RPS + _CH, _RPS - _CH)])
        plsc.subcore_barrier()

        @pl.loop(0, _ROUNDS)
        def _(r):
            row0 = ss * (_ROUNDS * _K) + r * _K
            wrow0 = ss * (_ROUNDS * (_CH // 16)) + r * (_CH // 16)
            pltpu.sync_copy(src_hbm.at[cc, pl.ds(row0, _K)], gidx_v)
            pltpu.sync_copy(dst_hbm.at[pl.ds(row0, _K)], didx_v)
            pltpu.sync_copy(w_hbm.at[pl.ds(wrow0, _CH // 16)], w_v)

            gathers = [
                pltpu.async_copy(t_hbm.at[gidx_v.at[j]],
                                 rows_v.at[pl.ds(j * 128, 128)], gsem)
                for j in range(_K)
            ]
            for c in gathers:
                c.wait()

            @pl.loop(0, _CH // 16)
            def _(g):
                w16 = w_v[g]
                for m in range(16):
                    e = g * 16 + m
                    rows_v[e] = rows_v[e] * _splat(w16, m)

            scatters = [
                pltpu.async_copy(rows_v.at[pl.ds(j * 128, 128)],
                                 acc.at[didx_v.at[j]], ssem, add=True)
                for j in range(_K)
            ]
            for c in scatters:
                c.wait()

        plsc.subcore_barrier()
        pltpu.sync_copy(acc.at[pl.ds(ss * _RPS, _RPS)],
                        out_hbm.at[pl.ds(cc * _N + ss * _RPS, _RPS)])

    return layer(t_in, src2, dst2, w2)


def _mean4(a, b, c, d):
    sh = (2 * _N * _H // 128, 128)
    args = [x.reshape(sh) for x in (a, b, c, d)]

    def body(a_ref, b_ref, c_ref, d_ref, o_ref):
        o_ref[...] = (a_ref[...] + b_ref[...] + c_ref[...] + d_ref[...]) * 0.25

    blk = (1000, 128)
    out = pl.pallas_call(
        body,
        out_shape=jax.ShapeDtypeStruct(sh, jnp.float32),
        grid=(sh[0] // blk[0],),
        in_specs=[pl.BlockSpec(blk, lambda i: (i, 0))] * 4,
        out_specs=pl.BlockSpec(blk, lambda i: (i, 0)),
    )(*args)
    return out.reshape(2 * _N, _H)


def kernel(emb_u, emb_i, edge_index, edge_weight):
    ego = jnp.concatenate([emb_u, emb_i], axis=0)
    t0 = ego.reshape(_N, 2, _H).transpose(1, 0, 2).reshape(2 * _N, _H)

    src = edge_index[0]
    dst = edge_index[1]
    pad = _EPAD - _E
    srcp = jnp.concatenate([src, jnp.zeros((pad,), jnp.int32)])
    dstp = jnp.concatenate([dst, jnp.zeros((pad,), jnp.int32)])
    wp = jnp.concatenate([edge_weight, jnp.zeros((pad,), jnp.float32)])
    src2 = jnp.stack([srcp, srcp + _N]).reshape(2, _EPAD // 128, 128)
    dst2 = dstp.reshape(_EPAD // 128, 128)
    w2 = wp.reshape(_EPAD // 16, 16)

    t1 = _sc_layer(t0, src2, dst2, w2)
    t2 = _sc_layer(t1, src2, dst2, w2)
    t3 = _sc_layer(t2, src2, dst2, w2)

    f = _mean4(t0, t1, t2, t3)
    f = f.reshape(2, _N, _H).transpose(1, 0, 2).reshape(_N, _D)
    return f[:_NU], f[_NU:]


# speedup vs baseline: 8.7027x; 8.7027x over previous
import functools

import jax
import jax.numpy as jnp
from jax import lax
from jax.experimental import pallas as pl
from jax.experimental.pallas import tpu as pltpu
from jax.experimental.pallas import tpu_sc as plsc

_N = 100000
_NP = 100096
_NU = 50000
_D = 32
_H = 16
_E = 1600000
_NSUB = 16
_CH = 1024
_K = _CH // 128
_ROUNDS = 100
_EPAD = _NSUB * _CH * _ROUNDS
_RPS = _NP // _NSUB


def _splat(vec, lane):
    idx = jnp.full((16, 1), lane, dtype=jnp.int32)
    dn = lax.GatherDimensionNumbers(
        offset_dims=(), collapsed_slice_dims=(0,), start_index_map=(0,))
    return lax.gather(vec, idx, dn, (1,),
                      mode=lax.GatherScatterMode.PROMISE_IN_BOUNDS)


def _sc_layer(t_in, src2, dst2, w2):
    mesh = plsc.VectorSubcoreMesh(core_axis_name="c", subcore_axis_name="s")

    @functools.partial(
        pl.kernel,
        out_type=jax.ShapeDtypeStruct((2 * _NP, _H), jnp.float32),
        mesh=mesh,
        scratch_types=[
            pltpu.VMEM((_K, 128), jnp.int32),
            pltpu.VMEM((_K, 128), jnp.int32),
            pltpu.VMEM((_CH // 16, 16), jnp.float32),
            pltpu.VMEM((_CH, _H), jnp.float32),
            pltpu.VMEM_SHARED((_NP, _H), jnp.float32),
            pltpu.SemaphoreType.DMA,
            pltpu.SemaphoreType.DMA,
        ],
        compiler_params=pltpu.CompilerParams(use_tc_tiling_on_sc=False),
    )
    def layer(t_hbm, src_hbm, dst_hbm, w_hbm, out_hbm,
              gidx_v, didx_v, w_v, rows_v, acc, gsem, ssem):
        cc = lax.axis_index("c")
        ss = lax.axis_index("s")

        zero = jnp.zeros((16,), jnp.float32)

        @pl.loop(0, _CH // 16)
        def _(g):
            for m in range(16):
                rows_v[g * 16 + m] = zero

        @pl.loop(0, _RPS // _CH)
        def _(i):
            pltpu.sync_copy(rows_v, acc.at[pl.ds(ss * _RPS + i * _CH, _CH)])

        rem = _RPS - (_RPS // _CH) * _CH
        pltpu.sync_copy(rows_v.at[pl.ds(0, rem)],
                        acc.at[pl.ds(ss * _RPS + _RPS - rem, rem)])
        plsc.subcore_barrier()

        @pl.loop(0, _ROUNDS)
        def _(r):
            row0 = ss * (_ROUNDS * _K) + r * _K
            wrow0 = ss * (_ROUNDS * (_CH // 16)) + r * (_CH // 16)
            pltpu.sync_copy(src_hbm.at[cc, pl.ds(row0, _K)], gidx_v)
            pltpu.sync_copy(dst_hbm.at[pl.ds(row0, _K)], didx_v)
            pltpu.sync_copy(w_hbm.at[pl.ds(wrow0, _CH // 16)], w_v)

            gathers = [
                pltpu.async_copy(t_hbm.at[gidx_v.at[j]],
                                 rows_v.at[pl.ds(j * 128, 128)], gsem)
                for j in range(_K)
            ]
            for c in gathers:
                c.wait()

            @pl.loop(0, _CH // 16)
            def _(g):
                w16 = w_v[g]
                for m in range(16):
                    e = g * 16 + m
                    rows_v[e] = rows_v[e] * _splat(w16, m)

            scatters = [
                pltpu.async_copy(rows_v.at[pl.ds(j * 128, 128)],
                                 acc.at[didx_v.at[j]], ssem, add=True)
                for j in range(_K)
            ]
            for c in scatters:
                c.wait()

        plsc.subcore_barrier()
        pltpu.sync_copy(acc.at[pl.ds(ss * _RPS, _RPS)],
                        out_hbm.at[pl.ds(cc * _NP + ss * _RPS, _RPS)])

    return layer(t_in, src2, dst2, w2)


def _mean4(a, b, c, d):
    sh = (2 * _NP * _H // 128, 128)
    args = [x.reshape(sh) for x in (a, b, c, d)]

    def body(a_ref, b_ref, c_ref, d_ref, o_ref):
        o_ref[...] = (a_ref[...] + b_ref[...] + c_ref[...] + d_ref[...]) * 0.25

    blk = (3128, 128)
    out = pl.pallas_call(
        body,
        out_shape=jax.ShapeDtypeStruct(sh, jnp.float32),
        grid=(sh[0] // blk[0],),
        in_specs=[pl.BlockSpec(blk, lambda i: (i, 0))] * 4,
        out_specs=pl.BlockSpec(blk, lambda i: (i, 0)),
    )(*args)
    return out.reshape(2 * _NP, _H)


def kernel(emb_u, emb_i, edge_index, edge_weight):
    ego = jnp.concatenate([emb_u, emb_i], axis=0)
    rowpad = jnp.zeros((_NP - _N, _H), jnp.float32)
    t0 = jnp.concatenate([ego[:, :_H], rowpad, ego[:, _H:], rowpad], axis=0)

    src = edge_index[0]
    dst = edge_index[1]
    pad = _EPAD - _E
    srcp = jnp.concatenate([src, jnp.zeros((pad,), jnp.int32)])
    dstp = jnp.concatenate([dst, jnp.zeros((pad,), jnp.int32)])
    wp = jnp.concatenate([edge_weight, jnp.zeros((pad,), jnp.float32)])
    src2 = jnp.stack([srcp, srcp + _NP]).reshape(2, _EPAD // 128, 128)
    dst2 = dstp.reshape(_EPAD // 128, 128)
    w2 = wp.reshape(_EPAD // 16, 16)

    t1 = _sc_layer(t0, src2, dst2, w2)
    t2 = _sc_layer(t1, src2, dst2, w2)
    t3 = _sc_layer(t2, src2, dst2, w2)

    f = _mean4(t0, t1, t2, t3)
    f = jnp.concatenate([f[:_N], f[_NP:_NP + _N]], axis=1)
    return f[:_NU], f[_NU:]


# speedup vs baseline: 12.4475x; 1.4303x over previous
import functools

import jax
import jax.numpy as jnp
from jax import lax
from jax.experimental import pallas as pl
from jax.experimental.pallas import tpu as pltpu
from jax.experimental.pallas import tpu_sc as plsc

_N = 100000
_NP = 100096
_NU = 50000
_D = 32
_H = 16
_E = 1600000
_NSUB = 16
_CH = 512
_K = _CH // 128
_R = 200
_EPAD = _NSUB * _CH * _R
_RPS = _NP // _NSUB
_GB = _CH // 16


def _splat(vec, lane):
    idx = jnp.full((16, 1), lane, dtype=jnp.int32)
    dn = lax.GatherDimensionNumbers(
        offset_dims=(), collapsed_slice_dims=(0,), start_index_map=(0,))
    return lax.gather(vec, idx, dn, (1,),
                      mode=lax.GatherScatterMode.PROMISE_IN_BOUNDS)


def _sc_forward(t0, src2, dst2, w2):
    mesh = plsc.VectorSubcoreMesh(core_axis_name="c", subcore_axis_name="s")
    tbl = jax.ShapeDtypeStruct((2 * _NP, _H), jnp.float32)

    @functools.partial(
        pl.kernel,
        out_type=(tbl, tbl, tbl),
        mesh=mesh,
        scratch_types=[
            pltpu.VMEM((2, _CH, _H), jnp.float32),
            pltpu.VMEM((4, _K, 128), jnp.int32),
            pltpu.VMEM((4, _K, 128), jnp.int32),
            pltpu.VMEM((4, _GB, 16), jnp.float32),
            pltpu.VMEM_SHARED((_NP, _H), jnp.float32),
            pltpu.SemaphoreType.DMA((2,)),
            pltpu.SemaphoreType.DMA((2,)),
            pltpu.SemaphoreType.DMA((4,)),
        ],
        compiler_params=pltpu.CompilerParams(use_tc_tiling_on_sc=False),
    )
    def fwd(t0_hbm, src_hbm, dst_hbm, w_hbm, o1_hbm, o2_hbm, o3_hbm,
            rows_v, gidx_v, didx_v, w_v, acc, gsem, ssem, isem):
        cc = lax.axis_index("c")
        ss = lax.axis_index("s")
        zero = jnp.zeros((16,), jnp.float32)
        ebase = ss * (_R * _K)
        wbase = ss * (_R * _GB)

        def fire_idx(r, q):
            row0 = ebase + r * _K
            wrow0 = wbase + r * _GB
            pltpu.async_copy(src_hbm.at[cc, pl.ds(row0, _K)],
                             gidx_v.at[q], isem.at[q])
            pltpu.async_copy(dst_hbm.at[pl.ds(row0, _K)],
                             didx_v.at[q], isem.at[q])
            pltpu.async_copy(w_hbm.at[pl.ds(wrow0, _GB)],
                             w_v.at[q], isem.at[q])

        def wait_idx(q):
            pltpu.make_async_copy(src_hbm.at[cc, pl.ds(0, _K)],
                                  gidx_v.at[q], isem.at[q]).wait()
            pltpu.make_async_copy(dst_hbm.at[pl.ds(0, _K)],
                                  didx_v.at[q], isem.at[q]).wait()
            pltpu.make_async_copy(w_hbm.at[pl.ds(0, _GB)],
                                  w_v.at[q], isem.at[q]).wait()

        def fire_gathers(t_hbm, b, q):
            for j in range(_K):
                pltpu.async_copy(t_hbm.at[gidx_v.at[q, j]],
                                 rows_v.at[b, pl.ds(j * 128, 128)], gsem.at[b])

        def wait_gathers(t_hbm, b):
            for j in range(_K):
                pltpu.make_async_copy(t_hbm.at[gidx_v.at[0, j]],
                                      rows_v.at[b, pl.ds(j * 128, 128)],
                                      gsem.at[b]).wait()

        def fire_scatters(b, q):
            for j in range(_K):
                pltpu.async_copy(rows_v.at[b, pl.ds(j * 128, 128)],
                                 acc.at[didx_v.at[q, j]], ssem.at[b], add=True)

        def wait_scatters(b):
            for j in range(_K):
                pltpu.make_async_copy(rows_v.at[b, pl.ds(j * 128, 128)],
                                      acc.at[didx_v.at[0, j]],
                                      ssem.at[b]).wait()

        def multiply(b, q):
            @pl.loop(0, _GB)
            def _(g):
                w16 = w_v[q, g]
                for m in range(16):
                    e = g * 16 + m
                    rows_v[b, e] = rows_v[b, e] * _splat(w16, m)

        def run_layer(t_hbm, out_hbm):
            @pl.loop(0, _GB)
            def _(g):
                for m in range(16):
                    rows_v[0, g * 16 + m] = zero

            @pl.loop(0, _RPS // _CH)
            def _(i):
                pltpu.sync_copy(rows_v.at[0],
                                acc.at[pl.ds(ss * _RPS + i * _CH, _CH)])

            rem = _RPS - (_RPS // _CH) * _CH
            pltpu.sync_copy(rows_v.at[0, pl.ds(0, rem)],
                            acc.at[pl.ds(ss * _RPS + _RPS - rem, rem)])
            plsc.subcore_barrier()

            pltpu.sync_copy(src_hbm.at[cc, pl.ds(ebase, _K)], gidx_v.at[0])
            pltpu.sync_copy(dst_hbm.at[pl.ds(ebase, _K)], didx_v.at[0])
            pltpu.sync_copy(w_hbm.at[pl.ds(wbase, _GB)], w_v.at[0])
            pltpu.sync_copy(src_hbm.at[cc, pl.ds(ebase + _K, _K)],
                            gidx_v.at[1])
            pltpu.sync_copy(dst_hbm.at[pl.ds(ebase + _K, _K)], didx_v.at[1])
            pltpu.sync_copy(w_hbm.at[pl.ds(wbase + _GB, _GB)], w_v.at[1])
            fire_gathers(t_hbm, 0, 0)

            def turn(r, u, fire_i=True, do_next=True, wait_sc=True,
                     wait_i=True):
                b = u % 2
                wait_gathers(t_hbm, b)
                if fire_i:
                    fire_idx(r + 2, (u + 2) % 4)
                if wait_sc:
                    wait_scatters(1 - b)
                if do_next:
                    if wait_i:
                        wait_idx((u + 1) % 4)
                    fire_gathers(t_hbm, 1 - b, (u + 1) % 4)
                multiply(b, u)
                fire_scatters(b, u)

            turn(0, 0, wait_sc=False, wait_i=False)
            for u in range(1, 4):
                turn(u, u)

            @pl.loop(1, _R // 4 - 1)
            def _(gi):
                for u in range(4):
                    turn(gi * 4 + u, u)

            turn(_R - 4, 0)
            turn(_R - 3, 1)
            turn(_R - 2, 2, fire_i=False)
            turn(_R - 1, 3, fire_i=False, do_next=False)

            wait_scatters((_R - 1) % 2)
            plsc.subcore_barrier()
            pltpu.sync_copy(acc.at[pl.ds(ss * _RPS, _RPS)],
                            out_hbm.at[pl.ds(cc * _NP + ss * _RPS, _RPS)])
            plsc.subcore_barrier()

        run_layer(t0_hbm, o1_hbm)
        run_layer(o1_hbm, o2_hbm)
        run_layer(o2_hbm, o3_hbm)

    return fwd(t0, src2, dst2, w2)


def _mean4(a, b, c, d):
    sh = (2 * _NP * _H // 128, 128)
    args = [x.reshape(sh) for x in (a, b, c, d)]

    def body(a_ref, b_ref, c_ref, d_ref, o_ref):
        o_ref[...] = (a_ref[...] + b_ref[...] + c_ref[...] + d_ref[...]) * 0.25

    blk = (3128, 128)
    out = pl.pallas_call(
        body,
        out_shape=jax.ShapeDtypeStruct(sh, jnp.float32),
        grid=(sh[0] // blk[0],),
        in_specs=[pl.BlockSpec(blk, lambda i: (i, 0))] * 4,
        out_specs=pl.BlockSpec(blk, lambda i: (i, 0)),
    )(*args)
    return out.reshape(2 * _NP, _H)


def kernel(emb_u, emb_i, edge_index, edge_weight):
    ego = jnp.concatenate([emb_u, emb_i], axis=0)
    rowpad = jnp.zeros((_NP - _N, _H), jnp.float32)
    t0 = jnp.concatenate([ego[:, :_H], rowpad, ego[:, _H:], rowpad], axis=0)

    src = edge_index[0]
    dst = edge_index[1]
    pad = _EPAD - _E
    srcp = jnp.concatenate([src, jnp.zeros((pad,), jnp.int32)])
    dstp = jnp.concatenate([dst, jnp.zeros((pad,), jnp.int32)])
    wp = jnp.concatenate([edge_weight, jnp.zeros((pad,), jnp.float32)])
    src2 = jnp.stack([srcp, srcp + _NP]).reshape(2, _EPAD // 128, 128)
    dst2 = dstp.reshape(_EPAD // 128, 128)
    w2 = wp.reshape(_EPAD // 16, 16)

    t1, t2, t3 = _sc_forward(t0, src2, dst2, w2)

    f = _mean4(t0, t1, t2, t3)
    f = jnp.concatenate([f[:_N], f[_NP:_NP + _N]], axis=1)
    return f[:_NU], f[_NU:]


# speedup vs baseline: 12.7927x; 1.0277x over previous
import functools

import jax
import jax.numpy as jnp
from jax import lax
from jax.experimental import pallas as pl
from jax.experimental.pallas import tpu as pltpu
from jax.experimental.pallas import tpu_sc as plsc

_N = 100000
_NP = 100096
_NU = 50000
_D = 32
_H = 16
_E = 1600000
_NSUB = 16
_CH = 512
_K = _CH // 128
_R = 200
_EPAD = _NSUB * _CH * _R
_RPS = _NP // _NSUB
_MC = 48


def _splat(vec, lane):
    idx = jnp.full((16, 1), lane, dtype=jnp.int32)
    dn = lax.GatherDimensionNumbers(
        offset_dims=(), collapsed_slice_dims=(0,), start_index_map=(0,))
    return lax.gather(vec, idx, dn, (1,),
                      mode=lax.GatherScatterMode.PROMISE_IN_BOUNDS)


def _sc_forward(t0, comb):
    mesh = plsc.VectorSubcoreMesh(core_axis_name="c", subcore_axis_name="s")
    tbl = jax.ShapeDtypeStruct((2 * _NP, _H), jnp.float32)
    ftab = jax.ShapeDtypeStruct((_N, _D), jnp.float32)

    @functools.partial(
        pl.kernel,
        out_type=(ftab, tbl, tbl),
        mesh=mesh,
        scratch_types=[
            pltpu.VMEM((2, _CH, _H), jnp.float32),
            pltpu.VMEM((4, 4 * _K, 128), jnp.int32),
            pltpu.VMEM_SHARED((_NP, _H), jnp.float32),
            pltpu.SemaphoreType.DMA((2,)),
            pltpu.SemaphoreType.DMA((2,)),
            pltpu.SemaphoreType.DMA((4,)),
        ],
        compiler_params=pltpu.CompilerParams(use_tc_tiling_on_sc=False,
                                             needs_layout_passes=False),
    )
    def fwd(t0_hbm, comb_hbm, f_hbm, o1_hbm, o2_hbm,
            rows_v, comb_v, acc, gsem, ssem, isem):
        cc = lax.axis_index("c")
        ss = lax.axis_index("s")
        zero = jnp.zeros((16,), jnp.float32)
        cbase = ss * (_R * 4 * _K)

        def fire_idx(r, q):
            pltpu.async_copy(comb_hbm.at[pl.ds(cbase + r * 4 * _K, 4 * _K)],
                             comb_v.at[q], isem.at[q])

        def wait_idx(q):
            pltpu.make_async_copy(comb_hbm.at[pl.ds(0, 4 * _K)],
                                  comb_v.at[q], isem.at[q]).wait()

        def fire_gathers(t_hbm, b, q):
            @pl.loop(0, _K)
            def _(j):
                pltpu.async_copy(t_hbm.at[comb_v.at[q, 4 * j + cc]],
                                 rows_v.at[b, pl.ds(j * 128, 128)], gsem.at[b])

        def wait_gathers(t_hbm, b):
            @pl.loop(0, _K)
            def _(j):
                pltpu.make_async_copy(t_hbm.at[comb_v.at[0, 4 * j]],
                                      rows_v.at[b, pl.ds(j * 128, 128)],
                                      gsem.at[b]).wait()

        def fire_scatters(b, q):
            for j in range(_K):
                pltpu.async_copy(rows_v.at[b, pl.ds(j * 128, 128)],
                                 acc.at[comb_v.at[q, 4 * j + 2]],
                                 ssem.at[b], add=True)

        def wait_scatters(b):
            for j in range(_K):
                pltpu.make_async_copy(rows_v.at[b, pl.ds(j * 128, 128)],
                                      acc.at[comb_v.at[0, 4 * j + 2]],
                                      ssem.at[b]).wait()

        def multiply(b, q):
            @pl.loop(0, _CH // 16)
            def _(g):
                j = g // 8
                g8 = g % 8
                wi = comb_v[q, 4 * j + 3, pl.ds(g8 * 16, 16)]
                w16 = plsc.bitcast(wi, jnp.float32)
                base = g * 16
                for m in range(16):
                    e = base + m
                    rows_v[b, e] = rows_v[b, e] * _splat(w16, m)

        def run_layer(t_hbm, out_hbm):
            @pl.loop(0, _CH // 16)
            def _(g):
                for m in range(16):
                    rows_v[0, g * 16 + m] = zero

            @pl.loop(0, _RPS // _CH)
            def _(i):
                pltpu.sync_copy(rows_v.at[0],
                                acc.at[pl.ds(ss * _RPS + i * _CH, _CH)])

            rem = _RPS - (_RPS // _CH) * _CH
            pltpu.sync_copy(rows_v.at[0, pl.ds(0, rem)],
                            acc.at[pl.ds(ss * _RPS + _RPS - rem, rem)])
            plsc.subcore_barrier()

            pltpu.sync_copy(comb_hbm.at[pl.ds(cbase, 4 * _K)], comb_v.at[0])
            pltpu.sync_copy(comb_hbm.at[pl.ds(cbase + 4 * _K, 4 * _K)],
                            comb_v.at[1])
            fire_gathers(t_hbm, 0, 0)

            def turn(r, u, fire_i=True, do_next=True, wait_sc=True,
                     wait_i=True):
                b = u % 2
                wait_gathers(t_hbm, b)
                if fire_i:
                    fire_idx(r + 2, (u + 2) % 4)
                if wait_sc:
                    wait_scatters(1 - b)
                if do_next:
                    if wait_i:
                        wait_idx((u + 1) % 4)
                    fire_gathers(t_hbm, 1 - b, (u + 1) % 4)
                multiply(b, u)
                fire_scatters(b, u)

            turn(0, 0, wait_sc=False, wait_i=False)
            for u in range(1, 4):
                turn(u, u)

            @pl.loop(1, _R // 4 - 1)
            def _(gi):
                for u in range(4):
                    turn(gi * 4 + u, u)

            turn(_R - 4, 0)
            turn(_R - 3, 1)
            turn(_R - 2, 2, fire_i=False)
            turn(_R - 1, 3, fire_i=False, do_next=False)

            wait_scatters((_R - 1) % 2)
            plsc.subcore_barrier()
            if out_hbm is not None:
                pltpu.sync_copy(acc.at[pl.ds(ss * _RPS, _RPS)],
                                out_hbm.at[pl.ds(cc * _NP + ss * _RPS, _RPS)])
                plsc.subcore_barrier()

        run_layer(t0_hbm, o1_hbm)
        run_layer(o1_hbm, o2_hbm)
        run_layer(o2_hbm, None)

        quarter = jnp.full((16,), 0.25, jnp.float32)

        def mean_fire_in(c, b):
            base = ss * _RPS + c * 128
            pltpu.async_copy(t0_hbm.at[pl.ds(cc * _NP + base, 128)],
                             rows_v.at[b, pl.ds(0, 128)], gsem.at[b])
            pltpu.async_copy(o1_hbm.at[pl.ds(cc * _NP + base, 128)],
                             rows_v.at[b, pl.ds(128, 128)], gsem.at[b])
            pltpu.async_copy(o2_hbm.at[pl.ds(cc * _NP + base, 128)],
                             rows_v.at[b, pl.ds(256, 128)], gsem.at[b])

        def mean_wait_in(b):
            for i in range(3):
                pltpu.make_async_copy(t0_hbm.at[pl.ds(0, 128)],
                                      rows_v.at[b, pl.ds(i * 128, 128)],
                                      gsem.at[b]).wait()

        def mean_compute(c, b):
            pltpu.sync_copy(acc.at[pl.ds(ss * _RPS + c * 128, 128)],
                            rows_v.at[b, pl.ds(384, 128)])
            mean_wait_in(b)

            @pl.loop(0, 8)
            def _(g8):
                base = g8 * 16
                for m in range(16):
                    e = base + m
                    s = (rows_v[b, e] + rows_v[b, 128 + e]
                         + rows_v[b, 256 + e] + rows_v[b, 384 + e])
                    rows_v[b, 384 + e] = s * quarter

            pltpu.async_copy(
                rows_v.at[b, pl.ds(384, 128)],
                f_hbm.at[pl.ds(ss * _RPS + c * 128, 128),
                         pl.ds(cc * _H, _H)], ssem.at[b])

        def mean_wait_out(b):
            pltpu.make_async_copy(rows_v.at[b, pl.ds(384, 128)],
                                  f_hbm.at[pl.ds(0, 128), pl.ds(0, _H)],
                                  ssem.at[b]).wait()

        mean_fire_in(0, 0)
        mean_fire_in(1, 1)
        mean_compute(0, 0)
        mean_fire_in(2, 0)
        mean_compute(1, 1)
        mean_fire_in(3, 1)

        @pl.loop(1, _MC // 2 - 1)
        def _(ci):
            for b in range(2):
                c = ci * 2 + b
                mean_wait_out(b)
                mean_compute(c, b)
                mean_fire_in(c + 2, b)

        mean_wait_out(0)
        mean_compute(_MC - 2, 0)
        mean_wait_out(1)
        mean_compute(_MC - 1, 1)
        mean_wait_out(0)
        mean_wait_out(1)

        def mean_rem(nrows):
            base = ss * _RPS + _MC * 128
            pltpu.sync_copy(t0_hbm.at[pl.ds(cc * _NP + base, nrows)],
                            rows_v.at[0, pl.ds(0, nrows)])
            pltpu.sync_copy(o1_hbm.at[pl.ds(cc * _NP + base, nrows)],
                            rows_v.at[0, pl.ds(128, nrows)])
            pltpu.sync_copy(o2_hbm.at[pl.ds(cc * _NP + base, nrows)],
                            rows_v.at[0, pl.ds(256, nrows)])
            pltpu.sync_copy(acc.at[pl.ds(base, nrows)],
                            rows_v.at[0, pl.ds(384, nrows)])

            @pl.loop(0, nrows // 16)
            def _(g8):
                bb = g8 * 16
                for m in range(16):
                    e = bb + m
                    s = (rows_v[0, e] + rows_v[0, 128 + e]
                         + rows_v[0, 256 + e] + rows_v[0, 384 + e])
                    rows_v[0, 384 + e] = s * quarter

            pltpu.sync_copy(rows_v.at[0, pl.ds(384, nrows)],
                            f_hbm.at[pl.ds(base, nrows), pl.ds(cc * _H, _H)])

        @pl.when(ss < _NSUB - 1)
        def _():
            mean_rem(112)

        @pl.when(ss == _NSUB - 1)
        def _():
            mean_rem(16)

    return fwd(t0, comb)


def kernel(emb_u, emb_i, edge_index, edge_weight):
    ego = jnp.concatenate([emb_u, emb_i], axis=0)
    rowpad = jnp.zeros((_NP - _N, _H), jnp.float32)
    t0 = jnp.concatenate([ego[:, :_H], rowpad, ego[:, _H:], rowpad], axis=0)

    src = edge_index[0]
    dst = edge_index[1]
    pad = _EPAD - _E
    srcp = jnp.concatenate([src, jnp.zeros((pad,), jnp.int32)])
    dstp = jnp.concatenate([dst, jnp.zeros((pad,), jnp.int32)])
    wp = jnp.concatenate([edge_weight, jnp.zeros((pad,), jnp.float32)])
    wb = lax.bitcast_convert_type(wp, jnp.int32)
    comb = jnp.concatenate(
        [srcp.reshape(-1, 1, 128), (srcp + _NP).reshape(-1, 1, 128),
         dstp.reshape(-1, 1, 128), wb.reshape(-1, 1, 128)],
        axis=1).reshape(-1, 128)

    f, _t1, _t2 = _sc_forward(t0, comb)
    return f[:_NU], f[_NU:]


# speedup vs baseline: 13.2012x; 1.0319x over previous
import functools

import jax
import jax.numpy as jnp
from jax import lax
from jax.experimental import pallas as pl
from jax.experimental.pallas import tpu as pltpu
from jax.experimental.pallas import tpu_sc as plsc

_N = 100000
_NP = 100096
_NU = 50000
_D = 32
_H = 16
_E = 1600000
_NSUB = 16
_CH = 640
_K = _CH // 128
_R = 160
_EPAD = _NSUB * _CH * _R
_RPS = _NP // _NSUB
_MC = 48


def _splat(vec, lane):
    idx = jnp.full((16, 1), lane, dtype=jnp.int32)
    dn = lax.GatherDimensionNumbers(
        offset_dims=(), collapsed_slice_dims=(0,), start_index_map=(0,))
    return lax.gather(vec, idx, dn, (1,),
                      mode=lax.GatherScatterMode.PROMISE_IN_BOUNDS)


def _sc_forward(t0, comb):
    mesh = plsc.VectorSubcoreMesh(core_axis_name="c", subcore_axis_name="s")
    tbl = jax.ShapeDtypeStruct((2 * _NP, _H), jnp.float32)
    ftab = jax.ShapeDtypeStruct((_N, _D), jnp.float32)

    @functools.partial(
        pl.kernel,
        out_type=(ftab, tbl, tbl),
        mesh=mesh,
        scratch_types=[
            pltpu.VMEM((2, _CH, _H), jnp.float32),
            pltpu.VMEM((4, 4 * _K, 128), jnp.int32),
            pltpu.VMEM_SHARED((_NP, _H), jnp.float32),
            pltpu.SemaphoreType.DMA((2,)),
            pltpu.SemaphoreType.DMA((2,)),
            pltpu.SemaphoreType.DMA((4,)),
        ],
        compiler_params=pltpu.CompilerParams(use_tc_tiling_on_sc=False,
                                             needs_layout_passes=False),
    )
    def fwd(t0_hbm, comb_hbm, f_hbm, o1_hbm, o2_hbm,
            rows_v, comb_v, acc, gsem, ssem, isem):
        cc = lax.axis_index("c")
        ss = lax.axis_index("s")
        zero = jnp.zeros((16,), jnp.float32)
        cbase = ss * (_R * 4 * _K)

        def fire_idx(r, q):
            pltpu.async_copy(comb_hbm.at[pl.ds(cbase + r * 4 * _K, 4 * _K)],
                             comb_v.at[q], isem.at[q])

        def wait_idx(q):
            pltpu.make_async_copy(comb_hbm.at[pl.ds(0, 4 * _K)],
                                  comb_v.at[q], isem.at[q]).wait()

        def fire_gathers(t_hbm, b, q):
            @pl.loop(0, _K)
            def _(j):
                pltpu.async_copy(t_hbm.at[comb_v.at[q, 4 * j + cc]],
                                 rows_v.at[b, pl.ds(j * 128, 128)], gsem.at[b])

        def wait_gathers(t_hbm, b):
            @pl.loop(0, _K)
            def _(j):
                pltpu.make_async_copy(t_hbm.at[comb_v.at[0, 4 * j]],
                                      rows_v.at[b, pl.ds(j * 128, 128)],
                                      gsem.at[b]).wait()

        def fire_scatters(b, q):
            for j in range(_K):
                pltpu.async_copy(rows_v.at[b, pl.ds(j * 128, 128)],
                                 acc.at[comb_v.at[q, 4 * j + 2]],
                                 ssem.at[b], add=True)

        def wait_scatters(b):
            for j in range(_K):
                pltpu.make_async_copy(rows_v.at[b, pl.ds(j * 128, 128)],
                                      acc.at[comb_v.at[0, 4 * j + 2]],
                                      ssem.at[b]).wait()

        def multiply(b, q):
            @pl.loop(0, _CH // 16)
            def _(g):
                j = g // 8
                g8 = g % 8
                wi = comb_v[q, 4 * j + 3, pl.ds(g8 * 16, 16)]
                w16 = plsc.bitcast(wi, jnp.float32)
                base = g * 16
                for m in range(16):
                    e = base + m
                    rows_v[b, e] = rows_v[b, e] * _splat(w16, m)

        def run_layer(t_hbm, out_hbm):
            @pl.loop(0, _CH // 16)
            def _(g):
                for m in range(16):
                    rows_v[0, g * 16 + m] = zero

            @pl.loop(0, _RPS // _CH)
            def _(i):
                pltpu.sync_copy(rows_v.at[0],
                                acc.at[pl.ds(ss * _RPS + i * _CH, _CH)])

            rem = _RPS - (_RPS // _CH) * _CH
            pltpu.sync_copy(rows_v.at[0, pl.ds(0, rem)],
                            acc.at[pl.ds(ss * _RPS + _RPS - rem, rem)])
            plsc.subcore_barrier()

            pltpu.sync_copy(comb_hbm.at[pl.ds(cbase, 4 * _K)], comb_v.at[0])
            pltpu.sync_copy(comb_hbm.at[pl.ds(cbase + 4 * _K, 4 * _K)],
                            comb_v.at[1])
            fire_gathers(t_hbm, 0, 0)

            def turn(r, u, fire_i=True, do_next=True, wait_sc=True,
                     wait_i=True):
                b = u % 2
                wait_gathers(t_hbm, b)
                if fire_i:
                    fire_idx(r + 2, (u + 2) % 4)
                if wait_sc:
                    wait_scatters(1 - b)
                if do_next:
                    if wait_i:
                        wait_idx((u + 1) % 4)
                    fire_gathers(t_hbm, 1 - b, (u + 1) % 4)
                multiply(b, u)
                fire_scatters(b, u)

            turn(0, 0, wait_sc=False, wait_i=False)
            for u in range(1, 4):
                turn(u, u)

            @pl.loop(1, _R // 4 - 1)
            def _(gi):
                for u in range(4):
                    turn(gi * 4 + u, u)

            turn(_R - 4, 0)
            turn(_R - 3, 1)
            turn(_R - 2, 2, fire_i=False)
            turn(_R - 1, 3, fire_i=False, do_next=False)

            wait_scatters((_R - 1) % 2)
            plsc.subcore_barrier()
            if out_hbm is not None:
                pltpu.sync_copy(acc.at[pl.ds(ss * _RPS, _RPS)],
                                out_hbm.at[pl.ds(cc * _NP + ss * _RPS, _RPS)])
                plsc.subcore_barrier()

        run_layer(t0_hbm, o1_hbm)
        run_layer(o1_hbm, o2_hbm)
        run_layer(o2_hbm, None)

        quarter = jnp.full((16,), 0.25, jnp.float32)

        def mean_fire_in(c, b):
            base = ss * _RPS + c * 128
            pltpu.async_copy(t0_hbm.at[pl.ds(cc * _NP + base, 128)],
                             rows_v.at[b, pl.ds(0, 128)], gsem.at[b])
            pltpu.async_copy(o1_hbm.at[pl.ds(cc * _NP + base, 128)],
                             rows_v.at[b, pl.ds(128, 128)], gsem.at[b])
            pltpu.async_copy(o2_hbm.at[pl.ds(cc * _NP + base, 128)],
                             rows_v.at[b, pl.ds(256, 128)], gsem.at[b])

        def mean_wait_in(b):
            for i in range(3):
                pltpu.make_async_copy(t0_hbm.at[pl.ds(0, 128)],
                                      rows_v.at[b, pl.ds(i * 128, 128)],
                                      gsem.at[b]).wait()

        def mean_compute(c, b):
            pltpu.sync_copy(acc.at[pl.ds(ss * _RPS + c * 128, 128)],
                            rows_v.at[b, pl.ds(384, 128)])
            mean_wait_in(b)

            @pl.loop(0, 8)
            def _(g8):
                base = g8 * 16
                for m in range(16):
                    e = base + m
                    s = (rows_v[b, e] + rows_v[b, 128 + e]
                         + rows_v[b, 256 + e] + rows_v[b, 384 + e])
                    rows_v[b, 384 + e] = s * quarter

            pltpu.async_copy(
                rows_v.at[b, pl.ds(384, 128)],
                f_hbm.at[pl.ds(ss * _RPS + c * 128, 128),
                         pl.ds(cc * _H, _H)], ssem.at[b])

        def mean_wait_out(b):
            pltpu.make_async_copy(rows_v.at[b, pl.ds(384, 128)],
                                  f_hbm.at[pl.ds(0, 128), pl.ds(0, _H)],
                                  ssem.at[b]).wait()

        mean_fire_in(0, 0)
        mean_fire_in(1, 1)
        mean_compute(0, 0)
        mean_fire_in(2, 0)
        mean_compute(1, 1)
        mean_fire_in(3, 1)

        @pl.loop(1, _MC // 2 - 1)
        def _(ci):
            for b in range(2):
                c = ci * 2 + b
                mean_wait_out(b)
                mean_compute(c, b)
                mean_fire_in(c + 2, b)

        mean_wait_out(0)
        mean_compute(_MC - 2, 0)
        mean_wait_out(1)
        mean_compute(_MC - 1, 1)
        mean_wait_out(0)
        mean_wait_out(1)

        def mean_rem(nrows):
            base = ss * _RPS + _MC * 128
            pltpu.sync_copy(t0_hbm.at[pl.ds(cc * _NP + base, nrows)],
                            rows_v.at[0, pl.ds(0, nrows)])
            pltpu.sync_copy(o1_hbm.at[pl.ds(cc * _NP + base, nrows)],
                            rows_v.at[0, pl.ds(128, nrows)])
            pltpu.sync_copy(o2_hbm.at[pl.ds(cc * _NP + base, nrows)],
                            rows_v.at[0, pl.ds(256, nrows)])
            pltpu.sync_copy(acc.at[pl.ds(base, nrows)],
                            rows_v.at[0, pl.ds(384, nrows)])

            @pl.loop(0, nrows // 16)
            def _(g8):
                bb = g8 * 16
                for m in range(16):
                    e = bb + m
                    s = (rows_v[0, e] + rows_v[0, 128 + e]
                         + rows_v[0, 256 + e] + rows_v[0, 384 + e])
                    rows_v[0, 384 + e] = s * quarter

            pltpu.sync_copy(rows_v.at[0, pl.ds(384, nrows)],
                            f_hbm.at[pl.ds(base, nrows), pl.ds(cc * _H, _H)])

        @pl.when(ss < _NSUB - 1)
        def _():
            mean_rem(112)

        @pl.when(ss == _NSUB - 1)
        def _():
            mean_rem(16)

    return fwd(t0, comb)


def kernel(emb_u, emb_i, edge_index, edge_weight):
    ego = jnp.concatenate([emb_u, emb_i], axis=0)
    rowpad = jnp.zeros((_NP - _N, _H), jnp.float32)
    t0 = jnp.concatenate([ego[:, :_H], rowpad, ego[:, _H:], rowpad], axis=0)

    src = edge_index[0]
    dst = edge_index[1]
    pad = _EPAD - _E
    srcp = jnp.concatenate([src, jnp.zeros((pad,), jnp.int32)])
    dstp = jnp.concatenate([dst, jnp.zeros((pad,), jnp.int32)])
    wp = jnp.concatenate([edge_weight, jnp.zeros((pad,), jnp.float32)])
    wb = lax.bitcast_convert_type(wp, jnp.int32)
    comb = jnp.concatenate(
        [srcp.reshape(-1, 1, 128), (srcp + _NP).reshape(-1, 1, 128),
         dstp.reshape(-1, 1, 128), wb.reshape(-1, 1, 128)],
        axis=1).reshape(-1, 128)

    f, _t1, _t2 = _sc_forward(t0, comb)
    return f[:_NU], f[_NU:]
